# baseline (speedup 1.0000x reference)
import functools

import jax
import jax.numpy as jnp
from jax import lax
from jax.experimental import pallas as pl
from jax.experimental.pallas import tpu as pltpu
from jax.experimental.pallas import tpu_sc as plsc

_K = 8
_C = 16
_QW = 16
_NG = 64
_GS = 64
_BIG = 1 << 30


def _tc_body(x_ref, t_ref, o_ref):
    x = x_ref[...]
    t = t_ref[...]
    ones_row = jnp.ones((1, x.shape[1]), jnp.float32)
    tn2 = lax.dot_general(
        ones_row, t * t, (((1,), (1,)), ((), ())),
        precision=lax.Precision.HIGHEST,
        preferred_element_type=jnp.float32)
    o_ref[...] = tn2 - 2.0 * lax.dot_general(
        x, t, (((1,), (1,)), ((), ())),
        precision=lax.Precision.HIGHEST,
        preferred_element_type=jnp.float32)


def _dist_rows(inputs, X_train):
    q, d = inputs.shape
    n = X_train.shape[0]
    bq = 128
    return pl.pallas_call(
        _tc_body,
        grid=(q // bq,),
        in_specs=[
            pl.BlockSpec((bq, d), lambda i: (i, 0)),
            pl.BlockSpec((n, d), lambda i: (0, 0)),
        ],
        out_specs=pl.BlockSpec((bq, n), lambda i: (i, 0)),
        out_shape=jax.ShapeDtypeStruct((q, n), jnp.float32),
    )(inputs, X_train)


def _sc_body(d2_hbm, y_hbm, out_hbm, d_ref, g_ref, idx_ref, rows_ref,
             obuf_ref, sem):
    w = lax.axis_index("s") * 2 + lax.axis_index("c")
    qbase = w * _QW
    lane = lax.broadcasted_iota(jnp.int32, (_QW,), 0)
    inf16 = jnp.full((_QW,), jnp.inf, jnp.float32)

    pltpu.sync_copy(d2_hbm.at[pl.ds(qbase, _QW), :], d_ref)

    def group_body(g, _):
        def acc_body(j, acc):
            col = jnp.full((_QW,), g * _GS + j, jnp.int32)
            return jnp.minimum(acc, plsc.load_gather(d_ref, [lane, col]))
        g_ref[g, :] = lax.fori_loop(0, _GS, acc_body, inf16)
        return 0
    lax.fori_loop(0, _NG, group_body, 0)

    for k in range(_K):
        def min_body(g, m):
            return jnp.minimum(m, g_ref[g, :])
        m = lax.fori_loop(0, _NG, min_body, inf16)

        def find_body(g, gf):
            eq = (g_ref[g, :] == m) & (gf == _NG)
            return jnp.where(eq, g, gf)
        g_found = lax.fori_loop(0, _NG, find_body,
                                jnp.full((_QW,), _NG, jnp.int32))

        base = g_found * _GS

        def scan_body(j, carry):
            col_found, gmin = carry
            col = base + j
            val = plsc.load_gather(d_ref, [lane, col])
            is_t = (val == m) & (col_found == _BIG)
            col_found = jnp.where(is_t, col, col_found)
            gmin = jnp.minimum(gmin, jnp.where(is_t, jnp.inf, val))
            return col_found, gmin
        col_found, gmin = lax.fori_loop(
            0, _GS, scan_body,
            (jnp.full((_QW,), _BIG, jnp.int32), inf16))

        plsc.store_scatter(d_ref, [lane, col_found], inf16)
        plsc.store_scatter(g_ref, [g_found, lane], gmin)
        plsc.store_scatter(idx_ref, [jnp.int32(k * _QW) + lane], col_found)

    pltpu.async_copy(y_hbm.at[idx_ref], rows_ref, sem).wait()

    for q in range(_QW):
        acc = rows_ref[q, :]
        for k in range(1, _K):
            acc = acc + rows_ref[k * _QW + q, :]
        p = acc * (1.0 / _K)
        for b in range(_C):
            obuf_ref[q, pl.ds(b * _C, _C)] = jnp.zeros((_C,), jnp.float32)
        qv = jnp.full((_QW,), q, jnp.int32)
        plsc.store_scatter(obuf_ref, [qv, lane * _C], 1.0 - p)
        plsc.store_scatter(obuf_ref, [qv, lane * _C + 1], p)

    pltpu.sync_copy(obuf_ref, out_hbm.at[pl.ds(qbase, _QW), :])


def _sc_topk_counts(d2, y_train):
    q, n = d2.shape
    mesh = plsc.VectorSubcoreMesh(core_axis_name="c", subcore_axis_name="s")
    f = functools.partial(
        pl.kernel,
        out_type=jax.ShapeDtypeStruct((q, _C * _C), jnp.float32),
        mesh=mesh,
        scratch_types=[
            pltpu.VMEM((_QW, n), jnp.float32),
            pltpu.VMEM((_NG, _QW), jnp.float32),
            pltpu.VMEM((_K * _QW,), jnp.int32),
            pltpu.VMEM((_K * _QW, _C), jnp.float32),
            pltpu.VMEM((_QW, _C * _C), jnp.float32),
            pltpu.SemaphoreType.DMA,
        ],
        compiler_params=pltpu.CompilerParams(use_tc_tiling_on_sc=False,
                                             needs_layout_passes=False),
    )(_sc_body)
    return f(d2, y_train)


def kernel(inputs, X_train, y_train):
    q = inputs.shape[0]
    d2 = _dist_rows(inputs, X_train)
    out2d = _sc_topk_counts(d2, y_train)
    return out2d.reshape(q, _C, _C)


# speedup vs baseline: 1.8020x; 1.8020x over previous
import functools

import jax
import jax.numpy as jnp
from jax import lax
from jax.experimental import pallas as pl
from jax.experimental.pallas import tpu as pltpu
from jax.experimental.pallas import tpu_sc as plsc

_K = 8
_C = 16
_QW = 16
_NG = 128
_GS = 32
_BIG = 1 << 30
_CN = 512


def _tc_body(x_ref, t_ref, o_ref, g_ref):
    x = x_ref[...]
    bq = x.shape[0]
    n = o_ref.shape[1]
    nchunks = n // _CN
    ones_row = jnp.ones((1, x.shape[1]), jnp.float32)
    gacc = jnp.full((bq, _NG), jnp.inf, jnp.float32)
    for c in range(nchunks):
        t = t_ref[pl.ds(c * _CN, _CN), :]
        tn2 = lax.dot_general(
            ones_row, t * t, (((1,), (1,)), ((), ())),
            precision=lax.Precision.HIGHEST,
            preferred_element_type=jnp.float32)
        d2c = tn2 - 2.0 * lax.dot_general(
            x, t, (((1,), (1,)), ((), ())),
            precision=lax.Precision.HIGHEST,
            preferred_element_type=jnp.float32)
        o_ref[:, pl.ds(c * _CN, _CN)] = d2c
        for s in range(_CN // _NG):
            gacc = jnp.minimum(gacc, d2c[:, s * _NG:(s + 1) * _NG])
    g_ref[...] = gacc


def _dist_and_pyramid(inputs, X_train):
    q, d = inputs.shape
    n = X_train.shape[0]
    bq = 128
    return pl.pallas_call(
        _tc_body,
        grid=(q // bq,),
        in_specs=[
            pl.BlockSpec((bq, d), lambda i: (i, 0)),
            pl.BlockSpec((n, d), lambda i: (0, 0)),
        ],
        out_specs=[
            pl.BlockSpec((bq, n), lambda i: (i, 0)),
            pl.BlockSpec((bq, _NG), lambda i: (i, 0)),
        ],
        out_shape=[
            jax.ShapeDtypeStruct((q, n), jnp.float32),
            jax.ShapeDtypeStruct((q, _NG), jnp.float32),
        ],
    )(inputs, X_train)


def _sc_body(d2_hbm, g_hbm, y_hbm, out_hbm, d_ref, gs_ref, gt_ref, idx_ref,
             rows_ref, obuf_ref, sem):
    w = lax.axis_index("s") * 2 + lax.axis_index("c")
    qbase = w * _QW
    lane = lax.broadcasted_iota(jnp.int32, (_QW,), 0)
    inf16 = jnp.full((_QW,), jnp.inf, jnp.float32)

    pltpu.sync_copy(d2_hbm.at[pl.ds(qbase, _QW), :], d_ref)
    pltpu.sync_copy(g_hbm.at[pl.ds(qbase, _QW), :], gs_ref)

    def tr_body(r, _):
        gt_ref[r, :] = plsc.load_gather(gs_ref, [lane, jnp.full((_QW,), r,
                                                                jnp.int32)])
        return 0
    lax.fori_loop(0, _NG, tr_body, 0, unroll=8)

    for k in range(_K):
        def argmin_body(r, carry):
            m, r_found = carry
            v = gt_ref[r, :]
            better = v < m
            return jnp.minimum(m, v), jnp.where(better, r, r_found)
        m, r_found = lax.fori_loop(
            0, _NG, argmin_body,
            (inf16, jnp.zeros((_QW,), jnp.int32)), unroll=8)

        def scan_body(c, carry):
            col_found, gmin = carry
            col = r_found + c * _NG
            val = plsc.load_gather(d_ref, [lane, col])
            is_t = (val == m) & (col_found == _BIG)
            col_found = jnp.where(is_t, col, col_found)
            gmin = jnp.minimum(gmin, jnp.where(is_t, jnp.inf, val))
            return col_found, gmin
        col_found, gmin = lax.fori_loop(
            0, _GS, scan_body,
            (jnp.full((_QW,), _BIG, jnp.int32), inf16), unroll=8)

        plsc.store_scatter(d_ref, [lane, col_found], inf16)
        plsc.store_scatter(gt_ref, [r_found, lane], gmin)
        plsc.store_scatter(idx_ref, [jnp.int32(k * _QW) + lane], col_found)

    pltpu.async_copy(y_hbm.at[idx_ref], rows_ref, sem).wait()

    for q in range(_QW):
        acc = rows_ref[q, :]
        for k in range(1, _K):
            acc = acc + rows_ref[k * _QW + q, :]
        p = acc * (1.0 / _K)
        for b in range(_C):
            obuf_ref[q, pl.ds(b * _C, _C)] = jnp.zeros((_C,), jnp.float32)
        qv = jnp.full((_QW,), q, jnp.int32)
        plsc.store_scatter(obuf_ref, [qv, lane * _C], 1.0 - p)
        plsc.store_scatter(obuf_ref, [qv, lane * _C + 1], p)

    pltpu.sync_copy(obuf_ref, out_hbm.at[pl.ds(qbase, _QW), :])


def _sc_topk_counts(d2, g, y_train):
    q, n = d2.shape
    mesh = plsc.VectorSubcoreMesh(core_axis_name="c", subcore_axis_name="s")
    f = functools.partial(
        pl.kernel,
        out_type=jax.ShapeDtypeStruct((q, _C * _C), jnp.float32),
        mesh=mesh,
        scratch_types=[
            pltpu.VMEM((_QW, n), jnp.float32),
            pltpu.VMEM((_QW, _NG), jnp.float32),
            pltpu.VMEM((_NG, _QW), jnp.float32),
            pltpu.VMEM((_K * _QW,), jnp.int32),
            pltpu.VMEM((_K * _QW, _C), jnp.float32),
            pltpu.VMEM((_QW, _C * _C), jnp.float32),
            pltpu.SemaphoreType.DMA,
        ],
        compiler_params=pltpu.CompilerParams(use_tc_tiling_on_sc=False,
                                             needs_layout_passes=False),
    )(_sc_body)
    return f(d2, g, y_train)


def kernel(inputs, X_train, y_train):
    q = inputs.shape[0]
    d2, g = _dist_and_pyramid(inputs, X_train)
    out2d = _sc_topk_counts(d2, g, y_train)
    return out2d.reshape(q, _C, _C)


# speedup vs baseline: 2.0298x; 1.1265x over previous
import functools

import jax
import jax.numpy as jnp
from jax import lax
from jax.experimental import pallas as pl
from jax.experimental.pallas import tpu as pltpu
from jax.experimental.pallas import tpu_sc as plsc

_K = 8
_C = 16
_QW = 16
_NG = 128
_GS = 32
_BIG = 1 << 30
_CN = 512


def _tc_body(x_ref, t_ref, o_ref, g_ref):
    x = x_ref[...]
    bq = x.shape[0]
    n = o_ref.shape[1]
    nchunks = n // _CN
    x_aug = jnp.concatenate(
        [x, jnp.full((bq, 1), -0.5, jnp.float32)], axis=1)
    t = t_ref[...]
    t_aug = jnp.concatenate(
        [t, jnp.sum(t * t, axis=1, keepdims=True)], axis=1)
    gacc = jnp.full((bq, _NG), jnp.inf, jnp.float32)
    for c in range(nchunks):
        d2c = -2.0 * lax.dot_general(
            x_aug, t_aug[c * _CN:(c + 1) * _CN, :], (((1,), (1,)), ((), ())),
            precision=lax.Precision.HIGHEST,
            preferred_element_type=jnp.float32)
        o_ref[:, pl.ds(c * _CN, _CN)] = d2c
        for s in range(_CN // _NG):
            gacc = jnp.minimum(gacc, d2c[:, s * _NG:(s + 1) * _NG])
    g_ref[...] = gacc


def _dist_and_pyramid(inputs, X_train):
    q, d = inputs.shape
    n = X_train.shape[0]
    bq = 256
    return pl.pallas_call(
        _tc_body,
        grid=(q // bq,),
        in_specs=[
            pl.BlockSpec((bq, d), lambda i: (i, 0)),
            pl.BlockSpec((n, d), lambda i: (0, 0)),
        ],
        out_specs=[
            pl.BlockSpec((bq, n), lambda i: (i, 0)),
            pl.BlockSpec((bq, _NG), lambda i: (i, 0)),
        ],
        out_shape=[
            jax.ShapeDtypeStruct((q, n), jnp.float32),
            jax.ShapeDtypeStruct((q, _NG), jnp.float32),
        ],
    )(inputs, X_train)


def _sc_body(d2_hbm, g_hbm, y_hbm, out_hbm, d_ref, gs_ref, gt_ref, idx_ref,
             rows_ref, obuf_ref, sem):
    w = lax.axis_index("s") * 2 + lax.axis_index("c")
    qbase = w * _QW
    lane = lax.broadcasted_iota(jnp.int32, (_QW,), 0)
    inf16 = jnp.full((_QW,), jnp.inf, jnp.float32)

    pltpu.sync_copy(d2_hbm.at[pl.ds(qbase, _QW), :], d_ref)
    pltpu.sync_copy(g_hbm.at[pl.ds(qbase, _QW), :], gs_ref)

    def tr_body(r, _):
        gt_ref[r, :] = plsc.load_gather(gs_ref, [lane, jnp.full((_QW,), r,
                                                                jnp.int32)])
        return 0
    lax.fori_loop(0, _NG, tr_body, 0, unroll=8)

    for k in range(_K):
        def argmin_body(r, carry):
            m, r_found = carry
            v = gt_ref[r, :]
            better = v < m
            return jnp.minimum(m, v), jnp.where(better, r, r_found)
        m, r_found = lax.fori_loop(
            0, _NG, argmin_body,
            (inf16, jnp.zeros((_QW,), jnp.int32)), unroll=8)

        def scan_body(c, carry):
            col_found, gmin = carry
            col = r_found + c * _NG
            val = plsc.load_gather(d_ref, [lane, col])
            is_t = (val == m) & (col_found == _BIG)
            col_found = jnp.where(is_t, col, col_found)
            gmin = jnp.minimum(gmin, jnp.where(is_t, jnp.inf, val))
            return col_found, gmin
        col_found, gmin = lax.fori_loop(
            0, _GS, scan_body,
            (jnp.full((_QW,), _BIG, jnp.int32), inf16), unroll=8)

        plsc.store_scatter(d_ref, [lane, col_found], inf16)
        plsc.store_scatter(gt_ref, [r_found, lane], gmin)
        plsc.store_scatter(idx_ref, [jnp.int32(k * _QW) + lane], col_found)

    pltpu.async_copy(y_hbm.at[idx_ref], rows_ref, sem).wait()

    for q in range(_QW):
        acc = rows_ref[q, :]
        for k in range(1, _K):
            acc = acc + rows_ref[k * _QW + q, :]
        p = acc * (1.0 / _K)
        for b in range(_C):
            obuf_ref[q, pl.ds(b * _C, _C)] = jnp.zeros((_C,), jnp.float32)
        qv = jnp.full((_QW,), q, jnp.int32)
        plsc.store_scatter(obuf_ref, [qv, lane * _C], 1.0 - p)
        plsc.store_scatter(obuf_ref, [qv, lane * _C + 1], p)

    pltpu.sync_copy(obuf_ref, out_hbm.at[pl.ds(qbase, _QW), :])


def _sc_topk_counts(d2, g, y_train):
    q, n = d2.shape
    mesh = plsc.VectorSubcoreMesh(core_axis_name="c", subcore_axis_name="s")
    f = functools.partial(
        pl.kernel,
        out_type=jax.ShapeDtypeStruct((q, _C * _C), jnp.float32),
        mesh=mesh,
        scratch_types=[
            pltpu.VMEM((_QW, n), jnp.float32),
            pltpu.VMEM((_QW, _NG), jnp.float32),
            pltpu.VMEM((_NG, _QW), jnp.float32),
            pltpu.VMEM((_K * _QW,), jnp.int32),
            pltpu.VMEM((_K * _QW, _C), jnp.float32),
            pltpu.VMEM((_QW, _C * _C), jnp.float32),
            pltpu.SemaphoreType.DMA,
        ],
        compiler_params=pltpu.CompilerParams(use_tc_tiling_on_sc=False,
                                             needs_layout_passes=False),
    )(_sc_body)
    return f(d2, g, y_train)


def kernel(inputs, X_train, y_train):
    q = inputs.shape[0]
    d2, g = _dist_and_pyramid(inputs, X_train)
    out2d = _sc_topk_counts(d2, g, y_train)
    return out2d.reshape(q, _C, _C)


# speedup vs baseline: 2.3497x; 1.1576x over previous
import functools

import jax
import jax.numpy as jnp
from jax import lax
from jax.experimental import pallas as pl
from jax.experimental.pallas import tpu as pltpu
from jax.experimental.pallas import tpu_sc as plsc

_K = 8
_C = 16
_QW = 16
_NG = 128
_GS = 32
_BIG = 1 << 30
_CN = 512


def _tc_body(x_ref, t_ref, o_ref, g_ref):
    x = x_ref[...]
    bq = x.shape[0]
    n = o_ref.shape[1]
    nchunks = n // _CN
    x_aug = jnp.concatenate(
        [x, jnp.full((bq, 1), -0.5, jnp.float32)], axis=1)
    t = t_ref[...]
    t_aug = jnp.concatenate(
        [t, jnp.sum(t * t, axis=1, keepdims=True)], axis=1)
    gacc = jnp.full((bq, _NG), jnp.inf, jnp.float32)
    for c in range(nchunks):
        d2c = -2.0 * lax.dot_general(
            x_aug, t_aug[c * _CN:(c + 1) * _CN, :], (((1,), (1,)), ((), ())),
            precision=lax.Precision.HIGHEST,
            preferred_element_type=jnp.float32)
        o_ref[:, pl.ds(c * _CN, _CN)] = d2c
        for s in range(_CN // _NG):
            gacc = jnp.minimum(gacc, d2c[:, s * _NG:(s + 1) * _NG])
    g_ref[...] = gacc


def _dist_and_pyramid(inputs, X_train):
    q, d = inputs.shape
    n = X_train.shape[0]
    bq = 256
    return pl.pallas_call(
        _tc_body,
        grid=(q // bq,),
        in_specs=[
            pl.BlockSpec((bq, d), lambda i: (i, 0)),
            pl.BlockSpec((n, d), lambda i: (0, 0)),
        ],
        out_specs=[
            pl.BlockSpec((bq, n), lambda i: (i, 0)),
            pl.BlockSpec((bq, _NG), lambda i: (i, 0)),
        ],
        out_shape=[
            jax.ShapeDtypeStruct((q, n), jnp.float32),
            jax.ShapeDtypeStruct((q, _NG), jnp.float32),
        ],
    )(inputs, X_train)


def _sc_body(d2_hbm, g_hbm, y2_hbm, out_hbm, d_ref, gs_ref, gt_ref, idx_ref,
             rows_ref, obuf_ref, sem):
    w = lax.axis_index("s") * 2 + lax.axis_index("c")
    qbase = w * _QW
    lane = lax.broadcasted_iota(jnp.int32, (_QW,), 0)
    inf16 = jnp.full((_QW,), jnp.inf, jnp.float32)

    pltpu.sync_copy(d2_hbm.at[pl.ds(qbase, _QW), :], d_ref)
    pltpu.sync_copy(g_hbm.at[pl.ds(qbase, _QW), :], gs_ref)

    def tr_body(r, _):
        gt_ref[r, :] = plsc.load_gather(gs_ref, [lane, jnp.full((_QW,), r,
                                                                jnp.int32)])
        return 0
    lax.fori_loop(0, _NG, tr_body, 0, unroll=8)

    offs = []
    for k in range(_K):
        def argmin_body(r, carry):
            m, r_found = carry
            v = gt_ref[r, :]
            better = v < m
            return jnp.minimum(m, v), jnp.where(better, r, r_found)
        m, r_found = lax.fori_loop(
            0, _NG, argmin_body,
            (inf16, jnp.zeros((_QW,), jnp.int32)), unroll=8)

        def scan_body(c, carry):
            col_found, gmin = carry
            col = r_found + c * _NG
            val = plsc.load_gather(d_ref, [lane, col])
            is_t = (val == m) & (col_found == _BIG)
            col_found = jnp.where(is_t, col, col_found)
            gmin = jnp.minimum(gmin, jnp.where(is_t, jnp.inf, val))
            return col_found, gmin
        col_found, gmin = lax.fori_loop(
            0, _GS, scan_body,
            (jnp.full((_QW,), _BIG, jnp.int32), inf16), unroll=8)

        plsc.store_scatter(d_ref, [lane, col_found], inf16)
        plsc.store_scatter(gt_ref, [r_found, lane], gmin)
        plsc.store_scatter(idx_ref, [jnp.int32(k * _QW) + lane],
                           col_found >> 3)
        offs.append((col_found & 7) << 4)

    pltpu.async_copy(y2_hbm.at[idx_ref], rows_ref, sem).wait()

    for q in range(_QW):
        for b in range(_C):
            obuf_ref[q, pl.ds(b * _C, _C)] = jnp.zeros((_C,), jnp.float32)
    for c in range(_C):
        acc = jnp.zeros((_QW,), jnp.float32)
        for k in range(_K):
            acc = acc + plsc.load_gather(
                rows_ref, [jnp.full((_QW,), k * _QW, jnp.int32) + lane,
                           offs[k] + c])
        p = acc * (1.0 / _K)
        cv = jnp.full((_QW,), c * _C, jnp.int32)
        plsc.store_scatter(obuf_ref, [lane, cv], 1.0 - p)
        plsc.store_scatter(obuf_ref, [lane, cv + 1], p)

    pltpu.sync_copy(obuf_ref, out_hbm.at[pl.ds(qbase, _QW), :])


def _sc_topk_counts(d2, g, y_train):
    q, n = d2.shape
    mesh = plsc.VectorSubcoreMesh(core_axis_name="c", subcore_axis_name="s")
    f = functools.partial(
        pl.kernel,
        out_type=jax.ShapeDtypeStruct((q, _C * _C), jnp.float32),
        mesh=mesh,
        scratch_types=[
            pltpu.VMEM((_QW, n), jnp.float32),
            pltpu.VMEM((_QW, _NG), jnp.float32),
            pltpu.VMEM((_NG, _QW), jnp.float32),
            pltpu.VMEM((_K * _QW,), jnp.int32),
            pltpu.VMEM((_K * _QW, 128), jnp.float32),
            pltpu.VMEM((_QW, _C * _C), jnp.float32),
            pltpu.SemaphoreType.DMA,
        ],
        compiler_params=pltpu.CompilerParams(use_tc_tiling_on_sc=True,
                                             needs_layout_passes=False),
    )(_sc_body)
    n_tr, c_tr = y_train.shape
    y2 = y_train.reshape(n_tr * c_tr // 128, 128)
    return f(d2, g, y2)


def kernel(inputs, X_train, y_train):
    q = inputs.shape[0]
    d2, g = _dist_and_pyramid(inputs, X_train)
    out2d = _sc_topk_counts(d2, g, y_train)
    return out2d.reshape(q, _C, _C)


# speedup vs baseline: 2.3901x; 1.0172x over previous
import functools

import jax
import jax.numpy as jnp
from jax import lax
from jax.experimental import pallas as pl
from jax.experimental.pallas import tpu as pltpu
from jax.experimental.pallas import tpu_sc as plsc

_K = 8
_C = 16
_QW = 16
_NG = 128
_GS = 32
_BIG = 1 << 30
_CN = 512


def _tc_body(x_ref, t_ref, o_ref, g_ref):
    x = x_ref[...]
    bq = x.shape[0]
    n = o_ref.shape[1]
    nchunks = n // _CN
    t = t_ref[...]
    ones_row = jnp.ones((1, x.shape[1]), jnp.float32)
    tn2 = lax.dot_general(
        ones_row, t * t, (((1,), (1,)), ((), ())),
        precision=lax.Precision.HIGHEST,
        preferred_element_type=jnp.float32)
    xh = x.astype(jnp.bfloat16)
    xl = (x - xh.astype(jnp.float32)).astype(jnp.bfloat16)
    th = t.astype(jnp.bfloat16)
    tl = (t - th.astype(jnp.float32)).astype(jnp.bfloat16)
    gacc = jnp.full((bq, _NG), jnp.inf, jnp.float32)
    dn = (((1,), (1,)), ((), ()))
    for c in range(nchunks):
        sl = slice(c * _CN, (c + 1) * _CN)
        xt = (lax.dot_general(xh, th[sl, :], dn,
                              preferred_element_type=jnp.float32)
              + lax.dot_general(xh, tl[sl, :], dn,
                                preferred_element_type=jnp.float32)
              + lax.dot_general(xl, th[sl, :], dn,
                                preferred_element_type=jnp.float32))
        d2c = tn2[:, sl] - 2.0 * xt
        o_ref[:, pl.ds(c * _CN, _CN)] = d2c
        for s in range(_CN // _NG):
            gacc = jnp.minimum(gacc, d2c[:, s * _NG:(s + 1) * _NG])
    g_ref[...] = gacc


def _dist_and_pyramid(inputs, X_train):
    q, d = inputs.shape
    n = X_train.shape[0]
    bq = 512
    return pl.pallas_call(
        _tc_body,
        grid=(q // bq,),
        in_specs=[
            pl.BlockSpec((bq, d), lambda i: (i, 0)),
            pl.BlockSpec((n, d), lambda i: (0, 0)),
        ],
        out_specs=[
            pl.BlockSpec((bq, n), lambda i: (i, 0)),
            pl.BlockSpec((bq, _NG), lambda i: (i, 0)),
        ],
        out_shape=[
            jax.ShapeDtypeStruct((q, n), jnp.float32),
            jax.ShapeDtypeStruct((q, _NG), jnp.float32),
        ],
    )(inputs, X_train)


def _sc_body(d2_hbm, g_hbm, y2_hbm, out_hbm, d_ref, gs_ref, gt_ref, idx_ref,
             rows_ref, obuf_ref, sem):
    w = lax.axis_index("s") * 2 + lax.axis_index("c")
    qbase = w * _QW
    lane = lax.broadcasted_iota(jnp.int32, (_QW,), 0)
    inf16 = jnp.full((_QW,), jnp.inf, jnp.float32)

    pltpu.sync_copy(d2_hbm.at[pl.ds(qbase, _QW), :], d_ref)
    pltpu.sync_copy(g_hbm.at[pl.ds(qbase, _QW), :], gs_ref)

    def tr_body(r, _):
        gt_ref[r, :] = plsc.load_gather(gs_ref, [lane, jnp.full((_QW,), r,
                                                                jnp.int32)])
        return 0
    lax.fori_loop(0, _NG, tr_body, 0, unroll=8)

    offs = []
    for k in range(_K):
        def argmin_body(r, carry):
            m, r_found = carry
            v = gt_ref[r, :]
            better = v < m
            return jnp.minimum(m, v), jnp.where(better, r, r_found)
        m, r_found = lax.fori_loop(
            0, _NG, argmin_body,
            (inf16, jnp.zeros((_QW,), jnp.int32)), unroll=8)

        def scan_body(c, carry):
            col_found, gmin = carry
            col = r_found + c * _NG
            val = plsc.load_gather(d_ref, [lane, col])
            is_t = (val == m) & (col_found == _BIG)
            col_found = jnp.where(is_t, col, col_found)
            gmin = jnp.minimum(gmin, jnp.where(is_t, jnp.inf, val))
            return col_found, gmin
        col_found, gmin = lax.fori_loop(
            0, _GS, scan_body,
            (jnp.full((_QW,), _BIG, jnp.int32), inf16), unroll=8)

        plsc.store_scatter(d_ref, [lane, col_found], inf16)
        plsc.store_scatter(gt_ref, [r_found, lane], gmin)
        plsc.store_scatter(idx_ref, [jnp.int32(k * _QW) + lane],
                           col_found >> 3)
        offs.append((col_found & 7) << 4)

    pltpu.async_copy(y2_hbm.at[idx_ref], rows_ref, sem).wait()

    for q in range(_QW):
        for b in range(_C):
            obuf_ref[q, pl.ds(b * _C, _C)] = jnp.zeros((_C,), jnp.float32)
    for c in range(_C):
        acc = jnp.zeros((_QW,), jnp.float32)
        for k in range(_K):
            acc = acc + plsc.load_gather(
                rows_ref, [jnp.full((_QW,), k * _QW, jnp.int32) + lane,
                           offs[k] + c])
        p = acc * (1.0 / _K)
        cv = jnp.full((_QW,), c * _C, jnp.int32)
        plsc.store_scatter(obuf_ref, [lane, cv], 1.0 - p)
        plsc.store_scatter(obuf_ref, [lane, cv + 1], p)

    pltpu.sync_copy(obuf_ref, out_hbm.at[pl.ds(qbase, _QW), :])


def _sc_topk_counts(d2, g, y_train):
    q, n = d2.shape
    mesh = plsc.VectorSubcoreMesh(core_axis_name="c", subcore_axis_name="s")
    f = functools.partial(
        pl.kernel,
        out_type=jax.ShapeDtypeStruct((q, _C * _C), jnp.float32),
        mesh=mesh,
        scratch_types=[
            pltpu.VMEM((_QW, n), jnp.float32),
            pltpu.VMEM((_QW, _NG), jnp.float32),
            pltpu.VMEM((_NG, _QW), jnp.float32),
            pltpu.VMEM((_K * _QW,), jnp.int32),
            pltpu.VMEM((_K * _QW, 128), jnp.float32),
            pltpu.VMEM((_QW, _C * _C), jnp.float32),
            pltpu.SemaphoreType.DMA,
        ],
        compiler_params=pltpu.CompilerParams(use_tc_tiling_on_sc=True,
                                             needs_layout_passes=False),
    )(_sc_body)
    n_tr, c_tr = y_train.shape
    y2 = y_train.reshape(n_tr * c_tr // 128, 128)
    return f(d2, g, y2)


def kernel(inputs, X_train, y_train):
    q = inputs.shape[0]
    d2, g = _dist_and_pyramid(inputs, X_train)
    out2d = _sc_topk_counts(d2, g, y_train)
    return out2d.reshape(q, _C, _C)


# speedup vs baseline: 2.5749x; 1.0773x over previous
import functools

import jax
import jax.numpy as jnp
from jax import lax
from jax.experimental import pallas as pl
from jax.experimental.pallas import tpu as pltpu
from jax.experimental.pallas import tpu_sc as plsc

_K = 8
_C = 16
_QW = 16
_NG = 128
_GS = 32
_BIG = 1 << 30
_CN = 512


def _tc_body(x_ref, t_ref, o_ref, g_ref):
    x = x_ref[...]
    bq = x.shape[0]
    n = o_ref.shape[1]
    nchunks = n // _CN
    t = t_ref[...]
    ones_row = jnp.ones((1, x.shape[1]), jnp.float32)
    tn2 = lax.dot_general(
        ones_row, t * t, (((1,), (1,)), ((), ())),
        precision=lax.Precision.HIGHEST,
        preferred_element_type=jnp.float32)
    xh = x.astype(jnp.bfloat16)
    xl = (x - xh.astype(jnp.float32)).astype(jnp.bfloat16)
    th = t.astype(jnp.bfloat16)
    tl = (t - th.astype(jnp.float32)).astype(jnp.bfloat16)
    gacc = jnp.full((bq, _NG), jnp.inf, jnp.float32)
    dn = (((1,), (1,)), ((), ()))
    for c in range(nchunks):
        sl = slice(c * _CN, (c + 1) * _CN)
        xt = (lax.dot_general(xh, th[sl, :], dn,
                              preferred_element_type=jnp.float32)
              + lax.dot_general(xh, tl[sl, :], dn,
                                preferred_element_type=jnp.float32)
              + lax.dot_general(xl, th[sl, :], dn,
                                preferred_element_type=jnp.float32))
        d2c = tn2[:, sl] - 2.0 * xt
        o_ref[:, pl.ds(c * _CN, _CN)] = d2c
        for s in range(_CN // _NG):
            gacc = jnp.minimum(gacc, d2c[:, s * _NG:(s + 1) * _NG])
    g_ref[...] = gacc


def _dist_and_pyramid(inputs, X_train):
    q, d = inputs.shape
    n = X_train.shape[0]
    bq = 512
    return pl.pallas_call(
        _tc_body,
        grid=(q // bq,),
        in_specs=[
            pl.BlockSpec((bq, d), lambda i: (i, 0)),
            pl.BlockSpec((n, d), lambda i: (0, 0)),
        ],
        out_specs=[
            pl.BlockSpec((bq, n), lambda i: (i, 0)),
            pl.BlockSpec((bq, _NG), lambda i: (i, 0)),
        ],
        out_shape=[
            jax.ShapeDtypeStruct((q, n), jnp.float32),
            jax.ShapeDtypeStruct((q, _NG), jnp.float32),
        ],
    )(inputs, X_train)


def _sc_body(d2_hbm, g_hbm, y2_hbm, out_hbm, d_ref, gs_ref, gt_ref, idx_ref,
             rows_ref, obuf_ref, sem, dsem):
    w = lax.axis_index("s") * 2 + lax.axis_index("c")
    qbase = w * _QW
    lane = lax.broadcasted_iota(jnp.int32, (_QW,), 0)
    inf16 = jnp.full((_QW,), jnp.inf, jnp.float32)

    with jax.named_scope("stage"):
        d2_cp = pltpu.async_copy(d2_hbm.at[pl.ds(qbase, _QW), :], d_ref,
                                 dsem)
        pltpu.sync_copy(g_hbm.at[pl.ds(qbase, _QW), :], gs_ref)

    with jax.named_scope("transpose"):
        def tr_body(r, col_v):
            gt_ref[r, :] = plsc.load_gather(gs_ref, [lane, col_v])
            return col_v + 1
        lax.fori_loop(0, _NG, tr_body, jnp.zeros((_QW,), jnp.int32),
                      unroll=8)

    with jax.named_scope("dwait"):
        d2_cp.wait()

    offs = []
    ycps = []
    for k in range(_K):
      with jax.named_scope("extract"):
        def argmin_body(r, carry):
            m, r_found = carry
            v = gt_ref[r, :]
            better = v < m
            return jnp.minimum(m, v), jnp.where(better, r, r_found)
        m, r_found = lax.fori_loop(
            0, _NG, argmin_body,
            (inf16, jnp.zeros((_QW,), jnp.int32)), unroll=8)

        def scan_body(c, carry):
            col_found, gmin = carry
            col = r_found + c * _NG
            val = plsc.load_gather(d_ref, [lane, col])
            is_t = (val == m) & (col_found == _BIG)
            col_found = jnp.where(is_t, col, col_found)
            gmin = jnp.minimum(gmin, jnp.where(is_t, jnp.inf, val))
            return col_found, gmin
        col_found, gmin = lax.fori_loop(
            0, _GS, scan_body,
            (jnp.full((_QW,), _BIG, jnp.int32), inf16), unroll=8)

        plsc.store_scatter(d_ref, [lane, col_found], inf16)
        plsc.store_scatter(gt_ref, [r_found, lane], gmin)
        plsc.store_scatter(idx_ref, [jnp.int32(k * _QW) + lane],
                           col_found >> 3)
        ycps.append(pltpu.async_copy(
            y2_hbm.at[idx_ref.at[pl.ds(k * _QW, _QW)]],
            rows_ref.at[pl.ds(k * _QW, _QW), :], sem))
      offs.append((col_found & 7) << 4)

    with jax.named_scope("counts"):
      for q in range(_QW):
        for b in range(_C):
            obuf_ref[q, pl.ds(b * _C, _C)] = jnp.zeros((_C,), jnp.float32)
      with jax.named_scope("ywait"):
        for cp in ycps:
            cp.wait()
      for c in range(_C):
        acc = jnp.zeros((_QW,), jnp.float32)
        for k in range(_K):
            acc = acc + plsc.load_gather(
                rows_ref, [jnp.full((_QW,), k * _QW, jnp.int32) + lane,
                           offs[k] + c])
        p = acc * (1.0 / _K)
        cv = jnp.full((_QW,), c * _C, jnp.int32)
        plsc.store_scatter(obuf_ref, [lane, cv], 1.0 - p)
        plsc.store_scatter(obuf_ref, [lane, cv + 1], p)
      del p, cv

    pltpu.sync_copy(obuf_ref, out_hbm.at[pl.ds(qbase, _QW), :])


def _sc_topk_counts(d2, g, y_train):
    q, n = d2.shape
    mesh = plsc.VectorSubcoreMesh(core_axis_name="c", subcore_axis_name="s")
    f = functools.partial(
        pl.kernel,
        out_type=jax.ShapeDtypeStruct((q, _C * _C), jnp.float32),
        mesh=mesh,
        scratch_types=[
            pltpu.VMEM((_QW, n), jnp.float32),
            pltpu.VMEM((_QW, _NG), jnp.float32),
            pltpu.VMEM((_NG, _QW), jnp.float32),
            pltpu.VMEM((_K * _QW,), jnp.int32),
            pltpu.VMEM((_K * _QW, 128), jnp.float32),
            pltpu.VMEM((_QW, _C * _C), jnp.float32),
            pltpu.SemaphoreType.DMA,
            pltpu.SemaphoreType.DMA,
        ],
        compiler_params=pltpu.CompilerParams(use_tc_tiling_on_sc=True,
                                             needs_layout_passes=False),
    )(_sc_body)
    n_tr, c_tr = y_train.shape
    y2 = y_train.reshape(n_tr * c_tr // 128, 128)
    return f(d2, g, y2)


def kernel(inputs, X_train, y_train):
    q = inputs.shape[0]
    d2, g = _dist_and_pyramid(inputs, X_train)
    out2d = _sc_topk_counts(d2, g, y_train)
    return out2d.reshape(q, _C, _C)


# speedup vs baseline: 2.6266x; 1.0201x over previous
import functools

import jax
import jax.numpy as jnp
from jax import lax
from jax.experimental import pallas as pl
from jax.experimental.pallas import tpu as pltpu
from jax.experimental.pallas import tpu_sc as plsc

_K = 8
_C = 16
_QW = 16
_NG = 128
_GS = 32
_BIG = 1 << 30
_CN = 512


def _tc_body(x_ref, t_ref, o_ref, g_ref):
    x = x_ref[...]
    bq = x.shape[0]
    n = o_ref.shape[1]
    nchunks = n // _CN
    t = t_ref[...]
    ones_row = jnp.ones((1, x.shape[1]), jnp.float32)
    tn2 = lax.dot_general(
        ones_row, t * t, (((1,), (1,)), ((), ())),
        precision=lax.Precision.HIGHEST,
        preferred_element_type=jnp.float32)
    xh = x.astype(jnp.bfloat16)
    xl = (x - xh.astype(jnp.float32)).astype(jnp.bfloat16)
    th = t.astype(jnp.bfloat16)
    tl = (t - th.astype(jnp.float32)).astype(jnp.bfloat16)
    gacc = jnp.full((bq, _NG), jnp.inf, jnp.float32)
    dn = (((1,), (1,)), ((), ()))
    for c in range(nchunks):
        sl = slice(c * _CN, (c + 1) * _CN)
        xt = (lax.dot_general(xh, th[sl, :], dn,
                              preferred_element_type=jnp.float32)
              + lax.dot_general(xh, tl[sl, :], dn,
                                preferred_element_type=jnp.float32)
              + lax.dot_general(xl, th[sl, :], dn,
                                preferred_element_type=jnp.float32))
        d2c = tn2[:, sl] - 2.0 * xt
        o_ref[:, pl.ds(c * _CN, _CN)] = d2c
        for s in range(_CN // _NG):
            gacc = jnp.minimum(gacc, d2c[:, s * _NG:(s + 1) * _NG])
    g_ref[...] = gacc


def _dist_and_pyramid(inputs, X_train):
    q, d = inputs.shape
    n = X_train.shape[0]
    bq = 512
    return pl.pallas_call(
        _tc_body,
        grid=(q // bq,),
        in_specs=[
            pl.BlockSpec((bq, d), lambda i: (i, 0)),
            pl.BlockSpec((n, d), lambda i: (0, 0)),
        ],
        out_specs=[
            pl.BlockSpec((bq, n), lambda i: (i, 0)),
            pl.BlockSpec((bq, _NG), lambda i: (i, 0)),
        ],
        out_shape=[
            jax.ShapeDtypeStruct((q, n), jnp.float32),
            jax.ShapeDtypeStruct((q, _NG), jnp.float32),
        ],
    )(inputs, X_train)


def _sc_body(d2_hbm, g_hbm, y2_hbm, out_hbm, d_ref, gs_ref, gt_ref, idx_ref,
             rows_ref, obuf_ref, sem, dsem):
    w = lax.axis_index("s") * 2 + lax.axis_index("c")
    qbase = w * _QW
    lane = lax.broadcasted_iota(jnp.int32, (_QW,), 0)
    inf16 = jnp.full((_QW,), jnp.inf, jnp.float32)

    d2_cp = pltpu.async_copy(d2_hbm.at[pl.ds(qbase, _QW), :], d_ref, dsem)
    pltpu.sync_copy(g_hbm.at[pl.ds(qbase, _QW), :], gs_ref)

    def tr_body(r, col_v):
        gt_ref[r, :] = plsc.load_gather(gs_ref, [lane, col_v])
        return col_v + 1
    lax.fori_loop(0, _NG, tr_body, jnp.zeros((_QW,), jnp.int32), unroll=8)

    d2_cp.wait()

    offs = []
    ycps = []
    for k in range(_K):
        def argmin_body(r, carry):
            m, r_found = carry
            v = gt_ref[r, :]
            better = v < m
            return jnp.minimum(m, v), jnp.where(better, r, r_found)
        m, r_found = lax.fori_loop(
            0, _NG, argmin_body,
            (inf16, jnp.zeros((_QW,), jnp.int32)), unroll=8)

        def scan_body(c, carry):
            col_found, gmin = carry
            col = r_found + c * _NG
            val = plsc.load_gather(d_ref, [lane, col])
            is_t = (val == m) & (col_found == _BIG)
            col_found = jnp.where(is_t, col, col_found)
            gmin = jnp.minimum(gmin, jnp.where(is_t, jnp.inf, val))
            return col_found, gmin
        col_found, gmin = lax.fori_loop(
            0, _GS, scan_body,
            (jnp.full((_QW,), _BIG, jnp.int32), inf16), unroll=8)

        plsc.store_scatter(d_ref, [lane, col_found], inf16)
        plsc.store_scatter(gt_ref, [r_found, lane], gmin)
        plsc.store_scatter(idx_ref, [jnp.int32(k * _QW) + lane],
                           col_found >> 3)
        ycps.append(pltpu.async_copy(
            y2_hbm.at[idx_ref.at[pl.ds(k * _QW, _QW)]],
            rows_ref.at[pl.ds(k * _QW, _QW), :], sem))
        offs.append((col_found & 7) << 4)

    for q in range(_QW):
        for b in range(_C):
            obuf_ref[q, pl.ds(b * _C, _C)] = jnp.zeros((_C,), jnp.float32)
    accs = [jnp.zeros((_QW,), jnp.float32) for _ in range(_C)]
    for k in range(_K):
        ycps[k].wait()
        rowv = jnp.full((_QW,), k * _QW, jnp.int32) + lane
        for c in range(_C):
            accs[c] = accs[c] + plsc.load_gather(rows_ref,
                                                 [rowv, offs[k] + c])
    for c in range(_C):
        p = accs[c] * (1.0 / _K)
        cv = jnp.full((_QW,), c * _C, jnp.int32)
        plsc.store_scatter(obuf_ref, [lane, cv], 1.0 - p)
        plsc.store_scatter(obuf_ref, [lane, cv + 1], p)

    pltpu.sync_copy(obuf_ref, out_hbm.at[pl.ds(qbase, _QW), :])


def _sc_topk_counts(d2, g, y_train):
    q, n = d2.shape
    mesh = plsc.VectorSubcoreMesh(core_axis_name="c", subcore_axis_name="s")
    f = functools.partial(
        pl.kernel,
        out_type=jax.ShapeDtypeStruct((q, _C * _C), jnp.float32),
        mesh=mesh,
        scratch_types=[
            pltpu.VMEM((_QW, n), jnp.float32),
            pltpu.VMEM((_QW, _NG), jnp.float32),
            pltpu.VMEM((_NG, _QW), jnp.float32),
            pltpu.VMEM((_K * _QW,), jnp.int32),
            pltpu.VMEM((_K * _QW, 128), jnp.float32),
            pltpu.VMEM((_QW, _C * _C), jnp.float32),
            pltpu.SemaphoreType.DMA,
            pltpu.SemaphoreType.DMA,
        ],
        compiler_params=pltpu.CompilerParams(use_tc_tiling_on_sc=True,
                                             needs_layout_passes=False),
    )(_sc_body)
    n_tr, c_tr = y_train.shape
    y2 = y_train.reshape(n_tr * c_tr // 128, 128)
    return f(d2, g, y2)


def kernel(inputs, X_train, y_train):
    q = inputs.shape[0]
    d2, g = _dist_and_pyramid(inputs, X_train)
    out2d = _sc_topk_counts(d2, g, y_train)
    return out2d.reshape(q, _C, _C)
